# baseline (speedup 1.0000x reference)
import functools

import jax
import jax.numpy as jnp
from jax import lax
from jax.experimental import pallas as pl
from jax.experimental.pallas import tpu as pltpu
from jax.experimental.pallas import tpu_sc as plsc

_L = 16


@functools.cache
def _make_sc_cumsum(R, C):
    info = plsc.get_sparse_core_info()
    NC, NS = info.num_cores, info.num_subcores
    NW = NC * NS
    rows_per_w = R // NW
    n_groups = rows_per_w // _L
    mesh = plsc.VectorSubcoreMesh(core_axis_name="c", subcore_axis_name="s")

    @functools.partial(
        pl.kernel,
        mesh=mesh,
        out_type=jax.ShapeDtypeStruct((R, C), jnp.float32),
        scratch_types=[pltpu.VMEM((_L, C), jnp.float32)],
        compiler_params=pltpu.CompilerParams(
            use_tc_tiling_on_sc=False, needs_layout_passes=False
        ),
    )
    def body(x_hbm, out_hbm, buf):
        wid = lax.axis_index("s") * NC + lax.axis_index("c")
        lane = lax.iota(jnp.int32, _L)

        def group(g, carry):
            base = wid * rows_per_w + g * _L
            pltpu.sync_copy(x_hbm.at[pl.ds(base, _L)], buf)

            def col(j, acc):
                jv = jnp.full((_L,), j, dtype=jnp.int32)
                v = plsc.load_gather(buf, [lane, jv])
                acc = acc + v
                plsc.store_scatter(buf, [lane, jv], acc)
                return acc

            lax.fori_loop(0, C, col, jnp.zeros((_L,), jnp.float32))
            pltpu.sync_copy(buf, out_hbm.at[pl.ds(base, _L)])
            return carry

        lax.fori_loop(0, n_groups, group, 0)

    return body


def kernel(x):
    B, S, C = x.shape
    xf = x.reshape(B * S, C)
    out = _make_sc_cumsum(B * S, C)(xf)
    return out.reshape(x.shape)
